# speedup vs baseline: 15.2017x; 15.2017x over previous
import functools

import jax
import jax.numpy as jnp
from jax import lax
from jax.experimental import pallas as pl
from jax.experimental.pallas import tpu as pltpu
from jax.experimental.pallas import tpu_sc as plsc

B = 2
N = 10000
E = 320000
D = 128
R = B * (N + E)
NC, NS = 2, 16
NW = NC * NS
W = 120
NWIN = R // W
WPT = (NWIN + NW - 1) // NW


def _gunpool_sc(xh, idx0, idx1):
    mesh = plsc.VectorSubcoreMesh(core_axis_name="c", subcore_axis_name="s")

    @functools.partial(
        pl.kernel,
        out_type=jax.ShapeDtypeStruct((R, D), jnp.float32),
        mesh=mesh,
        scratch_types=[
            pltpu.VMEM((W,), jnp.int32),
            pltpu.VMEM((W,), jnp.int32),
            pltpu.VMEM((W, D), jnp.float32),
            pltpu.VMEM((W, D), jnp.float32),
            pltpu.SemaphoreType.DMA,
            pltpu.SemaphoreType.DMA,
        ],
    )
    def k(x_hbm, i0_hbm, i1_hbm, out_hbm, i0_v, i1_v, buf0, buf1, sem0, sem1):
        wid = lax.axis_index("s") * NC + lax.axis_index("c")

        @pl.loop(0, WPT)
        def _(t):
            w = wid + t * NW

            @pl.when(w < NWIN)
            def _():
                base = w * W
                pltpu.sync_copy(i0_hbm.at[pl.ds(base, W)], i0_v)
                pltpu.sync_copy(i1_hbm.at[pl.ds(base, W)], i1_v)
                c0 = pltpu.async_copy(x_hbm.at[i0_v], buf0, sem0)
                c1 = pltpu.async_copy(x_hbm.at[i1_v], buf1, sem1)
                c0.wait()
                c1.wait()

                @pl.loop(0, W)
                def _(r):
                    @pl.loop(0, D, step=16)
                    def _(c):
                        plsc.addupdate(
                            buf0.at[r, pl.ds(c, 16)], buf1[r, pl.ds(c, 16)]
                        )

                pltpu.sync_copy(buf0, out_hbm.at[pl.ds(base, W)])

    return k(xh, idx0, idx1)


def kernel(inputs, unpool_idx):
    u0 = unpool_idx[:, 0].astype(jnp.int32)
    u1 = unpool_idx[:, 1].astype(jnp.int32)
    ar = jnp.arange(N, dtype=jnp.int32)
    idx0 = jnp.concatenate([ar, u0, ar + N, u0 + N])
    idx1 = jnp.concatenate([ar, u1, ar + N, u1 + N])
    xh = (inputs * 0.5).reshape(B * N, D)
    out = _gunpool_sc(xh, idx0, idx1)
    return out.reshape(B, N + E, D)


# speedup vs baseline: 26.7824x; 1.7618x over previous
import functools

import jax
import jax.numpy as jnp
from jax import lax
from jax.experimental import pallas as pl
from jax.experimental.pallas import tpu as pltpu
from jax.experimental.pallas import tpu_sc as plsc

B = 2
N = 10000
E = 320000
D = 128
R = B * (N + E)
NC, NS = 2, 16
NW = NC * NS
W = 120
WPT = (R + NW * W - 1) // (NW * W)
RPAD = NW * WPT * W
HPT = WPT // 2


def _gunpool_sc(xh, idx0, idx1):
    mesh = plsc.VectorSubcoreMesh(core_axis_name="c", subcore_axis_name="s")

    @functools.partial(
        pl.kernel,
        out_type=jax.ShapeDtypeStruct((R, D), jnp.float32),
        mesh=mesh,
        scratch_types=[
            pltpu.VMEM((WPT * W,), jnp.int32),
            pltpu.VMEM((WPT * W,), jnp.int32),
            pltpu.VMEM((W, D), jnp.float32),
            pltpu.VMEM((W, D), jnp.float32),
            pltpu.VMEM((W, D), jnp.float32),
            pltpu.VMEM((W, D), jnp.float32),
            pltpu.SemaphoreType.DMA,
            pltpu.SemaphoreType.DMA,
            pltpu.SemaphoreType.DMA,
            pltpu.SemaphoreType.DMA,
        ],
    )
    def k(x_hbm, i0_hbm, i1_hbm, out_hbm, i0_all, i1_all, b0a, b1a, b0b, b1b,
          sa0, sa1, sb0, sb1):
        wid = lax.axis_index("s") * NC + lax.axis_index("c")
        tile_base = wid * (WPT * W)

        pltpu.sync_copy(i0_hbm.at[pl.ds(tile_base, WPT * W)], i0_all)
        pltpu.sync_copy(i1_hbm.at[pl.ds(tile_base, WPT * W)], i1_all)

        def gather(s, d0, d1, s0, s1):
            pltpu.async_copy(x_hbm.at[i0_all.at[pl.ds(s * W, W)]], d0, s0)
            pltpu.async_copy(x_hbm.at[i1_all.at[pl.ds(s * W, W)]], d1, s1)

        def wait(s, d0, d1, s0, s1):
            pltpu.make_async_copy(x_hbm.at[i0_all.at[pl.ds(s * W, W)]], d0, s0).wait()
            pltpu.make_async_copy(x_hbm.at[i1_all.at[pl.ds(s * W, W)]], d1, s1).wait()

        def accum_store(s, d0, d1):
            @pl.loop(0, W)
            def _(r):
                for c in range(0, D, 16):
                    plsc.addupdate(d0.at[r, pl.ds(c, 16)], d1[r, pl.ds(c, 16)])

            base = tile_base + s * W

            @pl.when(base < R)
            def _():
                pltpu.sync_copy(d0, out_hbm.at[pl.ds(base, W)])

        gather(0, b0a, b1a, sa0, sa1)

        @pl.loop(0, HPT)
        def _(kk):
            s = 2 * kk
            gather(s + 1, b0b, b1b, sb0, sb1)
            wait(s, b0a, b1a, sa0, sa1)
            accum_store(s, b0a, b1a)

            @pl.when(kk < HPT - 1)
            def _():
                gather(s + 2, b0a, b1a, sa0, sa1)

            wait(s + 1, b0b, b1b, sb0, sb1)
            accum_store(s + 1, b0b, b1b)

    return k(xh, idx0, idx1)


def kernel(inputs, unpool_idx):
    u0 = unpool_idx[:, 0].astype(jnp.int32)
    u1 = unpool_idx[:, 1].astype(jnp.int32)
    ar = jnp.arange(N, dtype=jnp.int32)
    pad = jnp.zeros((RPAD - R,), jnp.int32)
    idx0 = jnp.concatenate([ar, u0, ar + N, u0 + N, pad])
    idx1 = jnp.concatenate([ar, u1, ar + N, u1 + N, pad])
    xh = (inputs * 0.5).reshape(B * N, D)
    out = _gunpool_sc(xh, idx0, idx1)
    return out.reshape(B, N + E, D)
